# Initial kernel scaffold; baseline (speedup 1.0000x reference)
import jax
import jax.numpy as jnp
from jax.experimental import pallas as pl


def kernel(pos, x, edge_index, W_node, b_node, W1, b1, g1, bt1, W2, b2, g2, bt2, W3, b3, g3, bt3, W4, b4):
    raise NotImplementedError("write your pallas kernel here")


# speedup vs baseline: 7.2115x; 7.2115x over previous
import functools

import jax
import jax.numpy as jnp
from jax import lax
from jax.experimental import pallas as pl
from jax.experimental.pallas import tpu as pltpu
from jax.experimental.pallas import tpu_sc as plsc

N = 10000
E = 160000
DIN = 256
DOUT = 256
EPS = 1e-5

T = 4096
HI = 32.0
INV = T / HI

NC = 2
NS = 16
NW = NC * NS

EPW_HI = 5008
EPW_LO = 4992
NV_LO = EPW_LO // 16

EPS_E = E // NS
CH = 80
SCH = 2000


def _sc_mesh():
    return plsc.VectorSubcoreMesh(core_axis_name="c", subcore_axis_name="s",
                                  num_cores=NC, num_subcores=NS)


def _a_body(x_ref, w_ref, b_ref, o_ref):
    y = jnp.dot(x_ref[...], w_ref[...], preferred_element_type=jnp.float32,
                precision=lax.Precision.HIGHEST)
    y = y + b_ref[...]
    o_ref[0] = y[:, :128]
    o_ref[1] = y[:, 128:]


def _x_proj(x, w, b):
    blk = 400
    return pl.pallas_call(
        _a_body,
        grid=(N // blk,),
        in_specs=[
            pl.BlockSpec((blk, DIN), lambda i: (i, 0)),
            pl.BlockSpec((DIN, DOUT), lambda i: (0, 0)),
            pl.BlockSpec((1, DOUT), lambda i: (0, 0)),
        ],
        out_specs=pl.BlockSpec((2, blk, 128), lambda i: (0, i, 0)),
        out_shape=jax.ShapeDtypeStruct((2, N, 128), jnp.float32),
    )(x, w, b.reshape(1, DOUT))


def _b_body(row_h, col_h, px_h, py_h, pz_h,
            d_out, cnt_out, dsum_out, d2sum_out, deg_out,
            px, py, pz, rowb, colb, db, cnt_v, dsum_v, d2sum_v, deg_v):
    cid = lax.axis_index("c")
    sid = lax.axis_index("s")
    wid = sid * NC + cid
    base = jnp.where(wid < 16, wid * EPW_HI,
                     16 * EPW_HI + (wid - 16) * EPW_LO)
    base = pl.multiple_of(base, 16)

    zf = jnp.zeros((16,), jnp.float32)

    def zero_hist(i, _):
        sl = pl.ds(i * 16, 16)
        cnt_v[sl] = zf
        dsum_v[sl] = zf
        d2sum_v[sl] = zf
        return 0

    lax.fori_loop(0, T // 16, zero_hist, 0)

    def zero_deg(i, _):
        deg_v[pl.ds(i * 16, 16)] = zf
        return 0

    lax.fori_loop(0, N // 16, zero_deg, 0)

    pltpu.sync_copy(px_h, px)
    pltpu.sync_copy(py_h, py)
    pltpu.sync_copy(pz_h, pz)

    pltpu.sync_copy(row_h.at[pl.ds(base, EPW_LO)], rowb.at[pl.ds(0, EPW_LO)])
    pltpu.sync_copy(col_h.at[pl.ds(base, EPW_LO)], colb.at[pl.ds(0, EPW_LO)])

    @pl.when(wid < 16)
    def _():
        pltpu.sync_copy(row_h.at[pl.ds(base + EPW_LO, 16)],
                        rowb.at[pl.ds(EPW_LO, 16)])
        pltpu.sync_copy(col_h.at[pl.ds(base + EPW_LO, 16)],
                        colb.at[pl.ds(EPW_LO, 16)])

    one = jnp.ones((16,), jnp.float32)

    def edge_vreg(j):
        sl = pl.ds(j * 16, 16)
        r = rowb[sl]
        c = colb[sl]
        dx = plsc.load_gather(px, [r]) - plsc.load_gather(px, [c])
        dy = plsc.load_gather(py, [r]) - plsc.load_gather(py, [c])
        dz = plsc.load_gather(pz, [r]) - plsc.load_gather(pz, [c])
        d2 = dx * dx + dy * dy + dz * dz
        iv = plsc.bitcast(d2, jnp.int32)
        iv = (iv >> 1) + 0x1FBD1DF5
        y = plsc.bitcast(iv, jnp.float32)
        y = 0.5 * (y + d2 / y)
        y = 0.5 * (y + d2 / y)
        y = 0.5 * (y + d2 / y)
        db[sl] = y
        b = jnp.clip((y * INV).astype(jnp.int32), 0, T - 1)
        plsc.addupdate_scatter(cnt_v, [b], one)
        plsc.addupdate_scatter(dsum_v, [b], y)
        plsc.addupdate_scatter(d2sum_v, [b], d2)
        plsc.addupdate_scatter(deg_v, [r], one)

    def loop_body(j, _):
        edge_vreg(j)
        return 0

    lax.fori_loop(0, NV_LO, loop_body, 0)

    @pl.when(wid < 16)
    def _():
        edge_vreg(NV_LO)

    pltpu.sync_copy(db.at[pl.ds(0, EPW_LO)], d_out.at[pl.ds(base, EPW_LO)])

    @pl.when(wid < 16)
    def _():
        pltpu.sync_copy(db.at[pl.ds(EPW_LO, 16)],
                        d_out.at[pl.ds(base + EPW_LO, 16)])

    toff = pl.multiple_of(wid * T, 8)
    noff = pl.multiple_of(wid * N, 8)
    pltpu.sync_copy(cnt_v, cnt_out.at[pl.ds(toff, T)])
    pltpu.sync_copy(dsum_v, dsum_out.at[pl.ds(toff, T)])
    pltpu.sync_copy(d2sum_v, d2sum_out.at[pl.ds(toff, T)])
    pltpu.sync_copy(deg_v, deg_out.at[pl.ds(noff, N)])


def _pass_b(row_h, col_h, px_h, py_h, pz_h):
    f = pl.kernel(
        _b_body,
        out_type=(
            jax.ShapeDtypeStruct((E,), jnp.float32),
            jax.ShapeDtypeStruct((NW * T,), jnp.float32),
            jax.ShapeDtypeStruct((NW * T,), jnp.float32),
            jax.ShapeDtypeStruct((NW * T,), jnp.float32),
            jax.ShapeDtypeStruct((NW * N,), jnp.float32),
        ),
        mesh=_sc_mesh(),
        compiler_params=pltpu.CompilerParams(needs_layout_passes=False),
        scratch_types=[
            pltpu.VMEM((N,), jnp.float32),
            pltpu.VMEM((N,), jnp.float32),
            pltpu.VMEM((N,), jnp.float32),
            pltpu.VMEM((EPW_HI,), jnp.int32),
            pltpu.VMEM((EPW_HI,), jnp.int32),
            pltpu.VMEM((EPW_HI,), jnp.float32),
            pltpu.VMEM((T,), jnp.float32),
            pltpu.VMEM((T,), jnp.float32),
            pltpu.VMEM((T,), jnp.float32),
            pltpu.VMEM((N,), jnp.float32),
        ],
    )
    return f(row_h, col_h, px_h, py_h, pz_h)


def _d_body(cntp, dsump, d2sump, degp,
            W1, b1, g1, bt1, W2, b2, g2, bt2, W3, b3, g3, bt3, W4, b4,
            dbar_o, psi_o, dinv_o):
    cnt = jnp.sum(cntp[...], axis=1, keepdims=True)
    dsum = jnp.sum(dsump[...], axis=1, keepdims=True)
    d2sum = jnp.sum(d2sump[...], axis=1, keepdims=True)
    fE = float(E)
    centers = (lax.broadcasted_iota(jnp.int32, (T, 1), 0).astype(jnp.float32)
               + 0.5) / INV
    dbar = jnp.where(cnt > 0, dsum / jnp.maximum(cnt, 1.0), centers)
    p = cnt / fE
    mean_d = jnp.sum(dsum) / fE
    var_d = jnp.sum(d2sum) / fE - mean_d * mean_d

    w1 = W1[...]
    h = dbar * w1 + b1[...]
    m1 = mean_d * w1 + b1[...]
    v1 = var_d * w1 * w1
    h = (h - m1) * lax.rsqrt(v1 + EPS) * g1[...] + bt1[...]
    h = jnp.where(h >= 0, h, 0.2 * h)

    h = jnp.dot(h, W2[...], preferred_element_type=jnp.float32,
                precision=lax.Precision.HIGHEST) + b2[...]
    m = jnp.sum(p * h, axis=0, keepdims=True)
    hc = h - m
    v = jnp.sum(p * hc * hc, axis=0, keepdims=True)
    h = hc * lax.rsqrt(v + EPS) * g2[...] + bt2[...]
    h = jnp.where(h >= 0, h, 0.2 * h)

    h = jnp.dot(h, W3[...], preferred_element_type=jnp.float32,
                precision=lax.Precision.HIGHEST) + b3[...]
    m = jnp.sum(p * h, axis=0, keepdims=True)
    hc = h - m
    v = jnp.sum(p * hc * hc, axis=0, keepdims=True)
    h = hc * lax.rsqrt(v + EPS) * g3[...] + bt3[...]
    h = jnp.where(h >= 0, h, 0.2 * h)

    psi = jnp.dot(h, W4[...], preferred_element_type=jnp.float32,
                  precision=lax.Precision.HIGHEST) + b4[...]

    dbar_o[...] = dbar
    psi_o[...] = psi

    deg = jnp.sum(degp[...], axis=1, keepdims=True)
    deg = jnp.where(deg == 0.0, 1.0, deg)
    dinv_o[...] = lax.rsqrt(deg)


def _build_tables(cntp, dsump, d2sump, degp,
                  W1, b1, g1, bt1, W2, b2, g2, bt2, W3, b3, g3, bt3, W4, b4):
    return pl.pallas_call(
        _d_body,
        out_shape=(
            jax.ShapeDtypeStruct((T, 1), jnp.float32),
            jax.ShapeDtypeStruct((T, 1), jnp.float32),
            jax.ShapeDtypeStruct((N, 1), jnp.float32),
        ),
    )(cntp, dsump, d2sump, degp,
      W1, b1.reshape(1, -1), g1.reshape(1, -1), bt1.reshape(1, -1),
      W2, b2.reshape(1, -1), g2.reshape(1, -1), bt2.reshape(1, -1),
      W3, b3.reshape(1, -1), g3.reshape(1, -1), bt3.reshape(1, -1),
      W4, b4.reshape(1, -1))


def _e_body(row_h, col_h, d_hbm, dbar_h, psi_h, dinv_h, xp_h, out_h,
            dinv_v, dbar_v, psi_v, rowb, colb, db, wb,
            srcbuf, dstbuf, gbuf, acc_ref, sem):
    cid = lax.axis_index("c")
    sid = lax.axis_index("s")
    base = pl.multiple_of(sid * EPS_E, 16)

    pltpu.sync_copy(dinv_h, dinv_v)
    pltpu.sync_copy(dbar_h, dbar_v)
    pltpu.sync_copy(psi_h, psi_v)

    zf = jnp.zeros((16,), jnp.float32)

    def zg(j, _):
        for k in range(8):
            gbuf[j, pl.ds(k * 16, 16)] = zf
        return 0

    lax.fori_loop(0, CH, zg, 0)

    @pl.when(sid < 10)
    def _():
        zbase = pl.multiple_of(sid * 1000, 8)
        for k in range(12):
            pltpu.sync_copy(gbuf, acc_ref.at[pl.ds(zbase + k * CH, CH)])
        pltpu.sync_copy(gbuf.at[pl.ds(0, 40)],
                        acc_ref.at[pl.ds(zbase + 960, 40)])
    plsc.subcore_barrier()

    noff = cid * N

    def super_chunk(sc_i, _):
        soff = pl.multiple_of(base + sc_i * SCH, 16)
        pltpu.sync_copy(row_h.at[pl.ds(soff, SCH)], rowb)
        pltpu.sync_copy(col_h.at[pl.ds(soff, SCH)], colb)
        pltpu.sync_copy(d_hbm.at[pl.ds(soff, SCH)], db)

        def wv_body(j, _):
            sl = pl.ds(j * 16, 16)
            r = rowb[sl]
            c = colb[sl]
            d = db[sl]
            b = jnp.clip((d * INV).astype(jnp.int32), 0, T - 1)
            db_b = plsc.load_gather(dbar_v, [b])
            go_r = d >= db_b
            li = jnp.clip(jnp.where(go_r, b, b - 1), 0, T - 1)
            ri = jnp.clip(jnp.where(go_r, b + 1, b), 0, T - 1)
            dl = plsc.load_gather(dbar_v, [li])
            dr = plsc.load_gather(dbar_v, [ri])
            pl_ = plsc.load_gather(psi_v, [li])
            pr = plsc.load_gather(psi_v, [ri])
            t = (d - dl) / jnp.maximum(dr - dl, 1e-12)
            psi_e = pl_ + t * (pr - pl_)
            w = plsc.load_gather(dinv_v, [r]) * plsc.load_gather(dinv_v, [c])
            wb[sl] = w * psi_e
            return 0

        lax.fori_loop(0, SCH // 16, wv_body, 0)

        def run_dir(srcb, dstb):
            def chunk(ch, _):
                off = pl.multiple_of(ch * CH, 16)

                def cp(i, _):
                    sl = pl.ds(i * 16, 16)
                    gs = pl.ds(off + i * 16, 16)
                    srcbuf[sl] = srcb[gs] + noff
                    dstbuf[sl] = dstb[gs]
                    return 0

                lax.fori_loop(0, CH // 16, cp, 0)
                pltpu.async_copy(xp_h.at[srcbuf], gbuf, sem).wait()

                def row(j, _):
                    widx = jnp.zeros((16,), jnp.int32) + (off + j)
                    wv = plsc.load_gather(wb, [widx])
                    for k in range(8):
                        sl = pl.ds(k * 16, 16)
                        gbuf[j, sl] = gbuf[j, sl] * wv
                    return 0

                lax.fori_loop(0, CH, row, 0)
                pltpu.sync_copy(gbuf, acc_ref.at[dstbuf], add=True)
                return 0

            lax.fori_loop(0, SCH // CH, chunk, 0)

        run_dir(colb, rowb)
        run_dir(rowb, colb)
        return 0

    lax.fori_loop(0, EPS_E // SCH, super_chunk, 0)

    plsc.subcore_barrier()

    @pl.when(sid < 10)
    def _():
        rows = 1000
        doff = pl.multiple_of(cid * N + sid * rows, 8)
        pltpu.sync_copy(acc_ref.at[pl.ds(sid * rows, rows)],
                        out_h.at[pl.ds(doff, rows)])


def _pass_e(row_h, col_h, d_hbm, dbar, psi, dinv, xp2):
    f = pl.kernel(
        _e_body,
        out_type=jax.ShapeDtypeStruct((2 * N, 128), jnp.float32),
        mesh=_sc_mesh(),
        compiler_params=pltpu.CompilerParams(needs_layout_passes=False),
        scratch_types=[
            pltpu.VMEM((N,), jnp.float32),
            pltpu.VMEM((T,), jnp.float32),
            pltpu.VMEM((T,), jnp.float32),
            pltpu.VMEM((SCH,), jnp.int32),
            pltpu.VMEM((SCH,), jnp.int32),
            pltpu.VMEM((SCH,), jnp.float32),
            pltpu.VMEM((SCH,), jnp.float32),
            pltpu.VMEM((CH,), jnp.int32),
            pltpu.VMEM((CH,), jnp.int32),
            pltpu.VMEM((CH, 128), jnp.float32),
            pltpu.VMEM_SHARED((N, 128), jnp.float32),
            pltpu.SemaphoreType.DMA,
        ],
    )
    xp_flat = xp2.reshape(2 * N, 128)
    return f(row_h, col_h, d_hbm, dbar, psi, dinv, xp_flat)


def _f_body(a_ref, pos_ref, o_ref):
    o_ref[:, :128] = jnp.maximum(a_ref[0], 0.0)
    o_ref[:, 128:256] = jnp.maximum(a_ref[1], 0.0)
    o_ref[:, 256:259] = pos_ref[...]


def _finalize(acc2, pos):
    blk = 400
    return pl.pallas_call(
        _f_body,
        grid=(N // blk,),
        in_specs=[
            pl.BlockSpec((2, blk, 128), lambda i: (0, i, 0)),
            pl.BlockSpec((blk, 3), lambda i: (i, 0)),
        ],
        out_specs=pl.BlockSpec((blk, 259), lambda i: (i, 0)),
        out_shape=jax.ShapeDtypeStruct((N, 259), jnp.float32),
    )(acc2, pos)


def kernel(pos, x, edge_index, W_node, b_node,
           W1, b1, g1, bt1, W2, b2, g2, bt2, W3, b3, g3, bt3, W4, b4):
    row_h = edge_index[0]
    col_h = edge_index[1]
    px_h, py_h, pz_h = pos[:, 0], pos[:, 1], pos[:, 2]
    xp2 = _x_proj(x, W_node, b_node)
    d_hbm, cntp, dsump, d2sump, degp = _pass_b(row_h, col_h, px_h, py_h, pz_h)
    dbar, psi, dinv = _build_tables(
        cntp.reshape(NW, T).T, dsump.reshape(NW, T).T,
        d2sump.reshape(NW, T).T, degp.reshape(NW, N).T,
        W1, b1, g1, bt1, W2, b2, g2, bt2, W3, b3, g3, bt3, W4, b4)
    acc2 = _pass_e(row_h, col_h, d_hbm,
                   dbar.reshape(T), psi.reshape(T), dinv.reshape(N), xp2)
    return _finalize(acc2.reshape(2, N, 128), pos)
